# Initial kernel scaffold; baseline (speedup 1.0000x reference)
import jax
import jax.numpy as jnp
from jax.experimental import pallas as pl


def kernel(x, edge_index, edge_attr, pka_labels, g1_Wi, g1_bi, g1_Wh, g1_bh, g1_Wo, g1_bo, g3_Wi, g3_bi, g3_Wh, g3_bh, g3_Wo, g3_bo, cls_W1, cls_b1, cls_W2, cls_b2, reg_W1, reg_b1, reg_W2, reg_b2):
    raise NotImplementedError("write your pallas kernel here")


# speedup vs baseline: 1.0290x; 1.0290x over previous
import functools

import jax
import jax.numpy as jnp
from jax import lax
from jax.experimental import pallas as pl
from jax.experimental.pallas import tpu as pltpu
from jax.experimental.pallas import tpu_sc as plsc

N_NODES = 10000
N_EDGES = 320000
D = 128
BOND = 16
DEPTH = 4

NC = 2
NS = 16
NW = NC * NS
EPW = N_EDGES // NW
BLK = 128
NFULL = EPW // BLK
TAIL = EPW - NFULL * BLK
NPW = 624
NREM = N_NODES - NS * NPW

BE = 512
GE = N_EDGES // BE
GPAD = N_EDGES + BE
BN = 1000
GN = N_NODES // BN

_mesh = functools.partial(
    plsc.VectorSubcoreMesh, core_axis_name="c", subcore_axis_name="s")


def _sc_gather_rows(table, idx):
    V = table.shape[0]

    def body(table_h, idx_h, out_h, idxb, rowb, idxt, rowt, sem):
        wid = lax.axis_index("s") * NC + lax.axis_index("c")
        base = wid * EPW

        def step(i, carry):
            b = base + i * BLK
            pltpu.sync_copy(idx_h.at[pl.ds(b, BLK)], idxb)
            pltpu.async_copy(table_h.at[idxb], rowb, sem).wait()
            pltpu.sync_copy(rowb, out_h.at[pl.ds(b, BLK)])
            return carry

        lax.fori_loop(0, NFULL, step, 0)
        b = base + NFULL * BLK
        pltpu.sync_copy(idx_h.at[pl.ds(b, TAIL)], idxt)
        pltpu.async_copy(table_h.at[idxt], rowt, sem).wait()
        pltpu.sync_copy(rowt, out_h.at[pl.ds(b, TAIL)])

    return pl.kernel(
        body,
        out_type=jax.ShapeDtypeStruct((N_EDGES, D), jnp.float32),
        mesh=_mesh(),
        scratch_types=[
            pltpu.VMEM((BLK,), jnp.int32),
            pltpu.VMEM((BLK, D), jnp.float32),
            pltpu.VMEM((TAIL,), jnp.int32),
            pltpu.VMEM((TAIL, D), jnp.float32),
            pltpu.SemaphoreType.DMA,
        ],
    )(table, idx)


def _sc_scatter_add(vals, dst, zeros_nd):
    def body(vals_h, dst_h, zeros_h, out_h, idxb, rowb, idxt, rowt, acc, sem):
        c = lax.axis_index("c")
        s = lax.axis_index("s")
        wid = s * NC + c
        r0 = s * NPW
        rr = NS * NPW
        pltpu.sync_copy(zeros_h.at[pl.ds(r0, NPW)], acc.at[pl.ds(r0, NPW)])

        @pl.when(s == NS - 1)
        def _():
            pltpu.sync_copy(zeros_h.at[pl.ds(rr, NREM)], acc.at[pl.ds(rr, NREM)])

        plsc.subcore_barrier()
        base = wid * EPW

        def step(i, carry):
            b = base + i * BLK
            pltpu.sync_copy(dst_h.at[pl.ds(b, BLK)], idxb)
            pltpu.sync_copy(vals_h.at[pl.ds(b, BLK)], rowb)
            pltpu.sync_copy(rowb, acc.at[idxb], add=True)
            return carry

        lax.fori_loop(0, NFULL, step, 0)
        b = base + NFULL * BLK
        pltpu.sync_copy(dst_h.at[pl.ds(b, TAIL)], idxt)
        pltpu.sync_copy(vals_h.at[pl.ds(b, TAIL)], rowt)
        pltpu.sync_copy(rowt, acc.at[idxt], add=True)
        plsc.subcore_barrier()
        pltpu.sync_copy(acc.at[pl.ds(r0, NPW)], out_h.at[c, pl.ds(r0, NPW)])

        @pl.when(s == NS - 1)
        def _():
            pltpu.sync_copy(acc.at[pl.ds(rr, NREM)],
                            out_h.at[c, pl.ds(rr, NREM)])

    return pl.kernel(
        body,
        out_type=jax.ShapeDtypeStruct((NC, N_NODES, D), jnp.float32),
        mesh=_mesh(),
        scratch_types=[
            pltpu.VMEM((BLK,), jnp.int32),
            pltpu.VMEM((BLK, D), jnp.float32),
            pltpu.VMEM((TAIL,), jnp.int32),
            pltpu.VMEM((TAIL, D), jnp.float32),
            pltpu.VMEM_SHARED((N_NODES, D), jnp.float32),
            pltpu.SemaphoreType.DMA,
        ],
    )(vals, dst, zeros_nd)


_TC_SEQ = pltpu.CompilerParams(dimension_semantics=("arbitrary",))


def _tc_node_matmul(xin, w_t):
    def body(x_r, w_r, o_r):
        o_r[...] = jnp.dot(x_r[...], w_r[...],
                           preferred_element_type=jnp.float32)

    return pl.pallas_call(
        body,
        grid=(GN,),
        in_specs=[
            pl.BlockSpec((BN, D), lambda i: (i, 0)),
            pl.BlockSpec((D, D), lambda i: (0, 0)),
        ],
        out_specs=pl.BlockSpec((BN, D), lambda i: (i, 0)),
        out_shape=jax.ShapeDtypeStruct((N_NODES, D), jnp.float32),
        compiler_params=_TC_SEQ,
    )(xin, w_t)


def _tc_edge_input(s0, ea, wie_t, bi, bh, wh_t):
    def body(s0_r, ea_r, wie_r, bi_r, bh_r, wh_r, h0b_r, g_r):
        i = pl.program_id(0)

        @pl.when(i < GE)
        def _():
            h0 = (s0_r[...]
                  + jnp.dot(ea_r[...], wie_r[...],
                            preferred_element_type=jnp.float32)
                  + bi_r[...])
            h0b_r[...] = h0 + bh_r[...]
            g_r[...] = jnp.dot(jax.nn.relu(h0), wh_r[...],
                               preferred_element_type=jnp.float32)

        @pl.when(i == GE)
        def _():
            g_r[...] = jnp.zeros_like(g_r)

    clamp = lambda i: (jnp.minimum(i, GE - 1), 0)
    return pl.pallas_call(
        body,
        grid=(GE + 1,),
        in_specs=[
            pl.BlockSpec((BE, D), clamp),
            pl.BlockSpec((BE, BOND), clamp),
            pl.BlockSpec((BOND, D), lambda i: (0, 0)),
            pl.BlockSpec((1, D), lambda i: (0, 0)),
            pl.BlockSpec((1, D), lambda i: (0, 0)),
            pl.BlockSpec((D, D), lambda i: (0, 0)),
        ],
        out_specs=[
            pl.BlockSpec((BE, D), clamp),
            pl.BlockSpec((BE, D), lambda i: (i, 0)),
        ],
        out_shape=[
            jax.ShapeDtypeStruct((N_EDGES, D), jnp.float32),
            jax.ShapeDtypeStruct((GPAD, D), jnp.float32),
        ],
        compiler_params=_TC_SEQ,
    )(s0, ea, wie_t, bi, bh, wh_t)


def _tc_partial_sum(partials):
    def body(p_r, o_r):
        o_r[...] = p_r[0] + p_r[1]

    return pl.pallas_call(
        body,
        grid=(GN,),
        in_specs=[pl.BlockSpec((NC, BN, D), lambda i: (0, i, 0))],
        out_specs=pl.BlockSpec((BN, D), lambda i: (i, 0)),
        out_shape=jax.ShapeDtypeStruct((N_NODES, D), jnp.float32),
        compiler_params=_TC_SEQ,
    )(partials)


def _tc_edge_iter(h0b, s, r, wh_t):
    def body(h0b_r, s_r, r_r, wh_r, g_r):
        i = pl.program_id(0)

        @pl.when(i < GE)
        def _():
            h = jax.nn.relu(h0b_r[...] + s_r[...] - r_r[...])
            g_r[...] = jnp.dot(h, wh_r[...],
                               preferred_element_type=jnp.float32)

        @pl.when(i == GE)
        def _():
            g_r[...] = jnp.zeros_like(g_r)

    clamp = lambda i: (jnp.minimum(i, GE - 1), 0)
    return pl.pallas_call(
        body,
        grid=(GE + 1,),
        in_specs=[
            pl.BlockSpec((BE, D), clamp),
            pl.BlockSpec((BE, D), clamp),
            pl.BlockSpec((BE, D), clamp),
            pl.BlockSpec((D, D), lambda i: (0, 0)),
        ],
        out_specs=pl.BlockSpec((BE, D), lambda i: (i, 0)),
        out_shape=jax.ShapeDtypeStruct((GPAD, D), jnp.float32),
        compiler_params=_TC_SEQ,
    )(h0b, s, r, wh_t)


def _tc_edge_last(h0b, s, r):
    def body(h0b_r, s_r, r_r, h_r):
        h_r[...] = jax.nn.relu(h0b_r[...] + s_r[...] - r_r[...])

    return pl.pallas_call(
        body,
        grid=(GE,),
        in_specs=[
            pl.BlockSpec((BE, D), lambda i: (i, 0)),
            pl.BlockSpec((BE, D), lambda i: (i, 0)),
            pl.BlockSpec((BE, D), lambda i: (i, 0)),
        ],
        out_specs=pl.BlockSpec((BE, D), lambda i: (i, 0)),
        out_shape=jax.ShapeDtypeStruct((N_EDGES, D), jnp.float32),
        compiler_params=_TC_SEQ,
    )(h0b, s, r)


def _tc_readout(partials_m, xin, wox_t, wom_t, bo):
    def body(pm_r, x_r, wox_r, wom_r, bo_r, o_r):
        magg = pm_r[0] + pm_r[1]
        iso = jnp.sum(magg, axis=1, keepdims=True) == 0.0
        m = jnp.where(iso, x_r[...], magg)
        o_r[...] = jax.nn.relu(
            jnp.dot(x_r[...], wox_r[...], preferred_element_type=jnp.float32)
            + jnp.dot(m, wom_r[...], preferred_element_type=jnp.float32)
            + bo_r[...])

    return pl.pallas_call(
        body,
        grid=(GN,),
        in_specs=[
            pl.BlockSpec((NC, BN, D), lambda i: (0, i, 0)),
            pl.BlockSpec((BN, D), lambda i: (i, 0)),
            pl.BlockSpec((D, D), lambda i: (0, 0)),
            pl.BlockSpec((D, D), lambda i: (0, 0)),
            pl.BlockSpec((1, D), lambda i: (0, 0)),
        ],
        out_specs=pl.BlockSpec((BN, D), lambda i: (i, 0)),
        out_shape=jax.ShapeDtypeStruct((N_NODES, D), jnp.float32),
        compiler_params=_TC_SEQ,
    )(partials_m, xin, wox_t, wom_t, bo)


def _tc_heads(h, w1_t, b1, w2_t8, b2_8, r1_t, rb1, r2_t8, rb2_8):
    def body(h_r, w1_r, b1_r, w2_r, b2_r, r1_r, rb1_r, r2_r, rb2_r,
             lg_r, pk_r, ls_r):
        i = pl.program_id(0)
        h1 = jax.nn.relu(
            jnp.dot(h_r[...], w1_r[...], preferred_element_type=jnp.float32)
            + b1_r[...])
        logits = jnp.dot(h1, w2_r[...],
                         preferred_element_type=jnp.float32) + b2_r[...]
        lg_r[...] = logits
        r1 = jax.nn.relu(
            jnp.dot(h_r[...], r1_r[...], preferred_element_type=jnp.float32)
            + rb1_r[...])
        pk_r[...] = jnp.dot(r1, r2_r[...],
                            preferred_element_type=jnp.float32) + rb2_r[...]
        d = logits[:, 1:2] - logits[:, 0:1]
        sp = jnp.maximum(d, 0.0) + jnp.log1p(jnp.exp(-jnp.abs(d)))

        @pl.when(i == 0)
        def _():
            ls_r[...] = jnp.zeros((1, 1), jnp.float32)

        ls_r[...] += jnp.sum(sp).reshape(1, 1)

    return pl.pallas_call(
        body,
        grid=(GN,),
        in_specs=[
            pl.BlockSpec((BN, D), lambda i: (i, 0)),
            pl.BlockSpec((D, D), lambda i: (0, 0)),
            pl.BlockSpec((1, D), lambda i: (0, 0)),
            pl.BlockSpec((D, 8), lambda i: (0, 0)),
            pl.BlockSpec((1, 8), lambda i: (0, 0)),
            pl.BlockSpec((D, D), lambda i: (0, 0)),
            pl.BlockSpec((1, D), lambda i: (0, 0)),
            pl.BlockSpec((D, 8), lambda i: (0, 0)),
            pl.BlockSpec((1, 8), lambda i: (0, 0)),
        ],
        out_specs=[
            pl.BlockSpec((BN, 8), lambda i: (i, 0)),
            pl.BlockSpec((BN, 8), lambda i: (i, 0)),
            pl.BlockSpec((1, 1), lambda i: (0, 0)),
        ],
        out_shape=[
            jax.ShapeDtypeStruct((N_NODES, 8), jnp.float32),
            jax.ShapeDtypeStruct((N_NODES, 8), jnp.float32),
            jax.ShapeDtypeStruct((1, 1), jnp.float32),
        ],
        compiler_params=_TC_SEQ,
    )(h, w1_t, b1, w2_t8, b2_8, r1_t, rb1, r2_t8, rb2_8)


def _mpnn_block(xin, src, dst, revz, ea, Wi, bi, Wh, bh, Wo, bo, zeros_nd):
    node_dim = xin.shape[1]
    wix_t = Wi[:, :node_dim].T
    wie_t = Wi[:, node_dim:].T
    wh_t = Wh.T
    wox_t = Wo[:, :node_dim].T
    wom_t = Wo[:, node_dim:].T
    bi2 = bi.reshape(1, D)
    bh2 = bh.reshape(1, D)
    bo2 = bo.reshape(1, D)

    xw = _tc_node_matmul(xin, wix_t)
    s0 = _sc_gather_rows(xw, src)
    h0b, gpad = _tc_edge_input(s0, ea, wie_t, bi2, bh2, wh_t)
    for k in range(DEPTH - 1):
        parts = _sc_scatter_add(gpad[:N_EDGES], dst, zeros_nd)
        pacc = _tc_partial_sum(parts)
        s = _sc_gather_rows(pacc, src)
        r = _sc_gather_rows(gpad, revz)
        if k < DEPTH - 2:
            gpad = _tc_edge_iter(h0b, s, r, wh_t)
        else:
            h_fin = _tc_edge_last(h0b, s, r)
    parts_m = _sc_scatter_add(h_fin, dst, zeros_nd)
    return _tc_readout(parts_m, xin, wox_t, wom_t, bo2)


def kernel(x, edge_index, edge_attr, pka_labels,
           g1_Wi, g1_bi, g1_Wh, g1_bh, g1_Wo, g1_bo,
           g3_Wi, g3_bi, g3_Wh, g3_bh, g3_Wo, g3_bo,
           cls_W1, cls_b1, cls_W2, cls_b2,
           reg_W1, reg_b1, reg_W2, reg_b2):
    src = edge_index[0]
    dst = edge_index[1]

    mult = jnp.max(edge_index).astype(jnp.int32) + 1
    keys = src.astype(jnp.int32) * mult + dst.astype(jnp.int32)
    rkeys = dst.astype(jnp.int32) * mult + src.astype(jnp.int32)
    order = jnp.argsort(keys, stable=True)
    sk = keys[order]
    pos = jnp.searchsorted(sk, rkeys)
    pos_c = jnp.clip(pos, 0, N_EDGES - 1)
    match = sk[pos_c] == rkeys
    rev = jnp.where(match, order[pos_c], -1).astype(jnp.int32)
    revz = jnp.where(rev < 0, N_EDGES, rev).astype(jnp.int32)

    zeros_nd = jnp.zeros((N_NODES, D), jnp.float32)

    h_static = _mpnn_block(x, src, dst, revz, edge_attr,
                           g1_Wi, g1_bi, g1_Wh, g1_bh, g1_Wo, g1_bo, zeros_nd)
    h_cur = _mpnn_block(h_static, src, dst, revz, edge_attr,
                        g3_Wi, g3_bi, g3_Wh, g3_bh, g3_Wo, g3_bo, zeros_nd)

    w2_t8 = jnp.zeros((D, 8), jnp.float32).at[:, :2].set(cls_W2.T)
    b2_8 = jnp.zeros((1, 8), jnp.float32).at[0, :2].set(cls_b2)
    r2_t8 = jnp.zeros((D, 8), jnp.float32).at[:, :1].set(reg_W2.T)
    rb2_8 = jnp.zeros((1, 8), jnp.float32).at[0, :1].set(reg_b2)

    logits8, pka8, lsum = _tc_heads(
        h_cur, cls_W1.T, cls_b1.reshape(1, D), w2_t8, b2_8,
        reg_W1.T, reg_b1.reshape(1, D), r2_t8, rb2_8)

    logits = logits8[:, :2]
    pka_raw = pka8[:, 0]
    loss_cla = lsum[0, 0] / N_NODES
    return (logits, pka_raw, 0.5 * loss_cla, loss_cla,
            jnp.array(0.0, jnp.float32))


# speedup vs baseline: 9.2511x; 8.9907x over previous
import functools

import jax
import jax.numpy as jnp
from jax import lax
from jax.experimental import pallas as pl
from jax.experimental.pallas import tpu as pltpu
from jax.experimental.pallas import tpu_sc as plsc

N_NODES = 10000
N_EDGES = 320000
D = 128
BOND = 16
DEPTH = 4

NC = 2
NS = 16
NW = NC * NS
EPW = N_EDGES // NW
BLK = 64
NSLOT = 2
NFULL = EPW // BLK
TAIL = EPW - NFULL * BLK
NGRP = NFULL // NSLOT
NSUP = NGRP // 2
NPW = 624
NREM = N_NODES - NS * NPW
CB = 16

BE = 512
GE = N_EDGES // BE
GPAD = N_EDGES + BE
BN = 1000
GN = N_NODES // BN

_mesh = functools.partial(
    plsc.VectorSubcoreMesh, core_axis_name="c", subcore_axis_name="s")


def _wid():
    return lax.axis_index("s") * NC + lax.axis_index("c")


def _stage_table(table_h, spm):
    s = lax.axis_index("s")
    r0 = s * NPW
    pltpu.sync_copy(table_h.at[pl.ds(r0, NPW)], spm.at[pl.ds(r0, NPW)])

    @pl.when(s == NS - 1)
    def _():
        rr = NS * NPW
        pltpu.sync_copy(table_h.at[pl.ds(rr, NREM)], spm.at[pl.ds(rr, NREM)])

    plsc.subcore_barrier()


def _emit_spmem_gather(spm, idxall, out_h, base, rows, sem_ga, sem_gb,
                       sem_sa, sem_sb):
    def fire_g(bank, g, sem):
        for b in range(NSLOT):
            blk = g * NSLOT + b
            idx = idxall.at[pl.ds(blk * BLK, BLK)]
            pltpu.async_copy(spm.at[idx], rows.at[bank * NSLOT + b], sem)

    def drain(sem, n, dst_is_row):
        for _ in range(n):
            if dst_is_row:
                pltpu.make_async_copy(out_h.at[pl.ds(base, BLK)],
                                      rows.at[0], sem).wait()
            else:
                pltpu.make_async_copy(rows.at[0],
                                      out_h.at[pl.ds(base, BLK)], sem).wait()

    def fire_s(bank, g, sem):
        for b in range(NSLOT):
            blk = g * NSLOT + b
            pltpu.async_copy(rows.at[bank * NSLOT + b],
                             out_h.at[pl.ds(base + blk * BLK, BLK)], sem)

    fire_g(0, 0, sem_ga)

    def step(k, carry):
        ga = 2 * k
        fire_g(1, ga + 1, sem_gb)
        drain(sem_ga, NSLOT, True)
        fire_s(0, ga, sem_sa)
        drain(sem_sa, NSLOT, False)

        @pl.when(k < NSUP - 1)
        def _():
            fire_g(0, ga + 2, sem_ga)

        drain(sem_gb, NSLOT, True)
        fire_s(1, ga + 1, sem_sb)
        drain(sem_sb, NSLOT, False)
        return carry

    lax.fori_loop(0, NSUP, step, 0)


def _sc_gather_spmem(table, idx):
    def body(table_h, idx_h, out_h, idxall, rows, idxt, rowt, spm,
             sem_ga, sem_gb, sem_sa, sem_sb, sem_t):
        _stage_table(table_h, spm)
        base = _wid() * EPW
        pltpu.sync_copy(idx_h.at[pl.ds(base, EPW)], idxall)
        _emit_spmem_gather(spm, idxall, out_h, base, rows,
                           sem_ga, sem_gb, sem_sa, sem_sb)
        b = base + NFULL * BLK
        pltpu.sync_copy(idx_h.at[pl.ds(b, TAIL)], idxt)
        pltpu.async_copy(spm.at[idxt], rowt, sem_t).wait()
        pltpu.sync_copy(rowt, out_h.at[pl.ds(b, TAIL)])

    return pl.kernel(
        body,
        out_type=jax.ShapeDtypeStruct((N_EDGES, D), jnp.float32),
        mesh=_mesh(),
        scratch_types=[
            pltpu.VMEM((EPW,), jnp.int32),
            pltpu.VMEM((2 * NSLOT, BLK, D), jnp.float32),
            pltpu.VMEM((TAIL,), jnp.int32),
            pltpu.VMEM((TAIL, D), jnp.float32),
            pltpu.VMEM_SHARED((N_NODES, D), jnp.float32),
            pltpu.SemaphoreType.DMA,
            pltpu.SemaphoreType.DMA,
            pltpu.SemaphoreType.DMA,
            pltpu.SemaphoreType.DMA,
            pltpu.SemaphoreType.DMA,
        ],
    )(table, idx)


def _sc_combine(pacc, src, gp, srcj, rlist, jlist, starts):
    def body(pacc_h, src_h, gp_h, srcj_h, rlist_h, jlist_h, starts_h, sr_h,
             idxall, rows, idxt, rowt, jbuf, sbuf, rbuf, bufs, bufr,
             st_vmem, spm, sem_ga, sem_gb, sem_sa, sem_sb, sem_t):
        _stage_table(pacc_h, spm)
        w = _wid()
        base = w * EPW
        pltpu.sync_copy(src_h.at[pl.ds(base, EPW)], idxall)
        _emit_spmem_gather(spm, idxall, sr_h, base, rows,
                           sem_ga, sem_gb, sem_sa, sem_sb)
        b = base + NFULL * BLK
        pltpu.sync_copy(src_h.at[pl.ds(b, TAIL)], idxt)
        pltpu.async_copy(spm.at[idxt], rowt, sem_t).wait()
        pltpu.sync_copy(rowt, sr_h.at[pl.ds(b, TAIL)])

        pltpu.sync_copy(starts_h, st_vmem)
        va = st_vmem[pl.ds(0, 16)]
        vb = st_vmem[pl.ds(16, 16)]
        vc = st_vmem[pl.ds(24, 16)]

        def entry(k):
            v = jnp.where(k == 32, vc[8], 0)
            for j in range(16):
                v = v + jnp.where(k == j, va[j], 0)
                v = v + jnp.where(k == 16 + j, vb[j], 0)
            return v

        c0 = entry(w) // CB
        c1 = (entry(w + 1) + (CB - 1)) // CB

        def chunk(c, carry):
            j0 = c * CB
            pltpu.sync_copy(jlist_h.at[pl.ds(j0, CB)], jbuf)
            pltpu.sync_copy(srcj_h.at[pl.ds(j0, CB)], sbuf)
            pltpu.sync_copy(rlist_h.at[pl.ds(j0, CB)], rbuf)
            pltpu.async_copy(spm.at[sbuf], bufs, sem_t).wait()
            pltpu.async_copy(gp_h.at[rbuf], bufr, sem_t).wait()

            for r in range(CB):
                for q in range(0, D, 16):
                    bufs[r, pl.ds(q, 16)] = (bufs[r, pl.ds(q, 16)]
                                             - bufr[r, pl.ds(q, 16)])
            pltpu.sync_copy(bufs, sr_h.at[jbuf])
            return carry

        lax.fori_loop(c0, c1, chunk, 0)

    return pl.kernel(
        body,
        out_type=jax.ShapeDtypeStruct((N_EDGES + 8, D), jnp.float32),
        mesh=_mesh(),
        scratch_types=[
            pltpu.VMEM((EPW,), jnp.int32),
            pltpu.VMEM((2 * NSLOT, BLK, D), jnp.float32),
            pltpu.VMEM((TAIL,), jnp.int32),
            pltpu.VMEM((TAIL, D), jnp.float32),
            pltpu.VMEM((CB,), jnp.int32),
            pltpu.VMEM((CB,), jnp.int32),
            pltpu.VMEM((CB,), jnp.int32),
            pltpu.VMEM((CB, D), jnp.float32),
            pltpu.VMEM((CB, D), jnp.float32),
            pltpu.VMEM((40,), jnp.int32),
            pltpu.VMEM_SHARED((N_NODES, D), jnp.float32),
            pltpu.SemaphoreType.DMA,
            pltpu.SemaphoreType.DMA,
            pltpu.SemaphoreType.DMA,
            pltpu.SemaphoreType.DMA,
            pltpu.SemaphoreType.DMA,
        ],
    )(pacc, src, gp, srcj, rlist, jlist, starts)


def _sc_scatter_add(vals, dst, zeros_nd):
    def body(vals_h, dst_h, zeros_h, out_h, rows, idxs, idxt, rowt, acc,
             sem_la, sem_lb, sem_w):
        c = lax.axis_index("c")
        s = lax.axis_index("s")
        _stage_table(zeros_h, acc)
        base = _wid() * EPW

        def fire_l(bank, g, sem):
            for b in range(NSLOT):
                blk = g * NSLOT + b
                slot = bank * NSLOT + b
                pltpu.async_copy(dst_h.at[pl.ds(base + blk * BLK, BLK)],
                                 idxs.at[slot], sem)
                pltpu.async_copy(vals_h.at[pl.ds(base + blk * BLK, BLK)],
                                 rows.at[slot], sem)

        def drain_l(sem):
            for _ in range(NSLOT):
                pltpu.make_async_copy(dst_h.at[pl.ds(base, BLK)],
                                      idxs.at[0], sem).wait()
                pltpu.make_async_copy(vals_h.at[pl.ds(base, BLK)],
                                      rows.at[0], sem).wait()

        def fire_w(bank):
            for b in range(NSLOT):
                slot = bank * NSLOT + b
                pltpu.async_copy(rows.at[slot], acc.at[idxs.at[slot]],
                                 sem_w, add=True)

        def drain_w():
            for _ in range(NSLOT):
                pltpu.make_async_copy(rows.at[0], acc.at[pl.ds(0, BLK)],
                                      sem_w).wait()

        fire_l(0, 0, sem_la)

        def step(k, carry):
            ga = 2 * k
            fire_l(1, ga + 1, sem_lb)
            drain_l(sem_la)
            fire_w(0)
            drain_w()

            @pl.when(k < NSUP - 1)
            def _():
                fire_l(0, ga + 2, sem_la)

            drain_l(sem_lb)
            fire_w(1)
            drain_w()
            return carry

        lax.fori_loop(0, NSUP, step, 0)
        b = base + NFULL * BLK
        pltpu.sync_copy(dst_h.at[pl.ds(b, TAIL)], idxt)
        pltpu.sync_copy(vals_h.at[pl.ds(b, TAIL)], rowt)
        pltpu.sync_copy(rowt, acc.at[idxt], add=True)
        plsc.subcore_barrier()
        r0 = s * NPW
        pltpu.sync_copy(acc.at[pl.ds(r0, NPW)], out_h.at[c, pl.ds(r0, NPW)])

        @pl.when(s == NS - 1)
        def _():
            rr = NS * NPW
            pltpu.sync_copy(acc.at[pl.ds(rr, NREM)],
                            out_h.at[c, pl.ds(rr, NREM)])

    return pl.kernel(
        body,
        out_type=jax.ShapeDtypeStruct((NC, N_NODES, D), jnp.float32),
        mesh=_mesh(),
        scratch_types=[
            pltpu.VMEM((2 * NSLOT, BLK, D), jnp.float32),
            pltpu.VMEM((2 * NSLOT, BLK), jnp.int32),
            pltpu.VMEM((TAIL,), jnp.int32),
            pltpu.VMEM((TAIL, D), jnp.float32),
            pltpu.VMEM_SHARED((N_NODES, D), jnp.float32),
            pltpu.SemaphoreType.DMA,
            pltpu.SemaphoreType.DMA,
            pltpu.SemaphoreType.DMA,
        ],
    )(vals, dst, zeros_nd)


_TC_SEQ = pltpu.CompilerParams(dimension_semantics=("arbitrary",))


def _tc_node_matmul(xin, w_t):
    def body(x_r, w_r, o_r):
        o_r[...] = jnp.dot(x_r[...], w_r[...],
                           preferred_element_type=jnp.float32)

    return pl.pallas_call(
        body,
        grid=(GN,),
        in_specs=[
            pl.BlockSpec((BN, D), lambda i: (i, 0)),
            pl.BlockSpec((D, D), lambda i: (0, 0)),
        ],
        out_specs=pl.BlockSpec((BN, D), lambda i: (i, 0)),
        out_shape=jax.ShapeDtypeStruct((N_NODES, D), jnp.float32),
        compiler_params=_TC_SEQ,
    )(xin, w_t)


def _tc_edge_input(s0, ea, wie_t, bi, bh, wh_t):
    def body(s0_r, ea_r, wie_r, bi_r, bh_r, wh_r, h0b_r, g_r):
        i = pl.program_id(0)

        @pl.when(i < GE)
        def _():
            h0 = (s0_r[...]
                  + jnp.dot(ea_r[...], wie_r[...],
                            preferred_element_type=jnp.float32)
                  + bi_r[...])
            h0b_r[...] = h0 + bh_r[...]
            g_r[...] = jnp.dot(jax.nn.relu(h0), wh_r[...],
                               preferred_element_type=jnp.float32)

        @pl.when(i == GE)
        def _():
            g_r[...] = jnp.zeros_like(g_r)

    clamp = lambda i: (jnp.minimum(i, GE - 1), 0)
    return pl.pallas_call(
        body,
        grid=(GE + 1,),
        in_specs=[
            pl.BlockSpec((BE, D), clamp),
            pl.BlockSpec((BE, BOND), clamp),
            pl.BlockSpec((BOND, D), lambda i: (0, 0)),
            pl.BlockSpec((1, D), lambda i: (0, 0)),
            pl.BlockSpec((1, D), lambda i: (0, 0)),
            pl.BlockSpec((D, D), lambda i: (0, 0)),
        ],
        out_specs=[
            pl.BlockSpec((BE, D), clamp),
            pl.BlockSpec((BE, D), lambda i: (i, 0)),
        ],
        out_shape=[
            jax.ShapeDtypeStruct((N_EDGES, D), jnp.float32),
            jax.ShapeDtypeStruct((GPAD, D), jnp.float32),
        ],
        compiler_params=_TC_SEQ,
    )(s0, ea, wie_t, bi, bh, wh_t)


def _tc_partial_sum(partials):
    def body(p_r, o_r):
        o_r[...] = p_r[0] + p_r[1]

    return pl.pallas_call(
        body,
        grid=(GN,),
        in_specs=[pl.BlockSpec((NC, BN, D), lambda i: (0, i, 0))],
        out_specs=pl.BlockSpec((BN, D), lambda i: (i, 0)),
        out_shape=jax.ShapeDtypeStruct((N_NODES, D), jnp.float32),
        compiler_params=_TC_SEQ,
    )(partials)


def _tc_edge_iter(h0b, sr, wh_t):
    def body(h0b_r, sr_r, wh_r, g_r):
        i = pl.program_id(0)

        @pl.when(i < GE)
        def _():
            h = jax.nn.relu(h0b_r[...] + sr_r[...])
            g_r[...] = jnp.dot(h, wh_r[...],
                               preferred_element_type=jnp.float32)

        @pl.when(i == GE)
        def _():
            g_r[...] = jnp.zeros_like(g_r)

    clamp = lambda i: (jnp.minimum(i, GE - 1), 0)
    return pl.pallas_call(
        body,
        grid=(GE + 1,),
        in_specs=[
            pl.BlockSpec((BE, D), clamp),
            pl.BlockSpec((BE, D), clamp),
            pl.BlockSpec((D, D), lambda i: (0, 0)),
        ],
        out_specs=pl.BlockSpec((BE, D), lambda i: (i, 0)),
        out_shape=jax.ShapeDtypeStruct((GPAD, D), jnp.float32),
        compiler_params=_TC_SEQ,
    )(h0b, sr, wh_t)


def _tc_edge_last(h0b, sr):
    def body(h0b_r, sr_r, h_r):
        h_r[...] = jax.nn.relu(h0b_r[...] + sr_r[...])

    return pl.pallas_call(
        body,
        grid=(GE,),
        in_specs=[
            pl.BlockSpec((BE, D), lambda i: (i, 0)),
            pl.BlockSpec((BE, D), lambda i: (i, 0)),
        ],
        out_specs=pl.BlockSpec((BE, D), lambda i: (i, 0)),
        out_shape=jax.ShapeDtypeStruct((N_EDGES, D), jnp.float32),
        compiler_params=_TC_SEQ,
    )(h0b, sr)


def _tc_readout(partials_m, xin, wox_t, wom_t, bo):
    def body(pm_r, x_r, wox_r, wom_r, bo_r, o_r):
        magg = pm_r[0] + pm_r[1]
        iso = jnp.sum(magg, axis=1, keepdims=True) == 0.0
        m = jnp.where(iso, x_r[...], magg)
        o_r[...] = jax.nn.relu(
            jnp.dot(x_r[...], wox_r[...], preferred_element_type=jnp.float32)
            + jnp.dot(m, wom_r[...], preferred_element_type=jnp.float32)
            + bo_r[...])

    return pl.pallas_call(
        body,
        grid=(GN,),
        in_specs=[
            pl.BlockSpec((NC, BN, D), lambda i: (0, i, 0)),
            pl.BlockSpec((BN, D), lambda i: (i, 0)),
            pl.BlockSpec((D, D), lambda i: (0, 0)),
            pl.BlockSpec((D, D), lambda i: (0, 0)),
            pl.BlockSpec((1, D), lambda i: (0, 0)),
        ],
        out_specs=pl.BlockSpec((BN, D), lambda i: (i, 0)),
        out_shape=jax.ShapeDtypeStruct((N_NODES, D), jnp.float32),
        compiler_params=_TC_SEQ,
    )(partials_m, xin, wox_t, wom_t, bo)


def _tc_heads(h, w1_t, b1, w2_t8, b2_8, r1_t, rb1, r2_t8, rb2_8):
    def body(h_r, w1_r, b1_r, w2_r, b2_r, r1_r, rb1_r, r2_r, rb2_r,
             lg_r, pk_r, ls_r):
        i = pl.program_id(0)
        h1 = jax.nn.relu(
            jnp.dot(h_r[...], w1_r[...], preferred_element_type=jnp.float32)
            + b1_r[...])
        logits = jnp.dot(h1, w2_r[...],
                         preferred_element_type=jnp.float32) + b2_r[...]
        lg_r[...] = logits
        r1 = jax.nn.relu(
            jnp.dot(h_r[...], r1_r[...], preferred_element_type=jnp.float32)
            + rb1_r[...])
        pk_r[...] = jnp.dot(r1, r2_r[...],
                            preferred_element_type=jnp.float32) + rb2_r[...]
        d = logits[:, 1:2] - logits[:, 0:1]
        sp = jnp.maximum(d, 0.0) + jnp.log1p(jnp.exp(-jnp.abs(d)))

        @pl.when(i == 0)
        def _():
            ls_r[...] = jnp.zeros((1, 1), jnp.float32)

        ls_r[...] += jnp.sum(sp).reshape(1, 1)

    return pl.pallas_call(
        body,
        grid=(GN,),
        in_specs=[
            pl.BlockSpec((BN, D), lambda i: (i, 0)),
            pl.BlockSpec((D, D), lambda i: (0, 0)),
            pl.BlockSpec((1, D), lambda i: (0, 0)),
            pl.BlockSpec((D, 8), lambda i: (0, 0)),
            pl.BlockSpec((1, 8), lambda i: (0, 0)),
            pl.BlockSpec((D, D), lambda i: (0, 0)),
            pl.BlockSpec((1, D), lambda i: (0, 0)),
            pl.BlockSpec((D, 8), lambda i: (0, 0)),
            pl.BlockSpec((1, 8), lambda i: (0, 0)),
        ],
        out_specs=[
            pl.BlockSpec((BN, 8), lambda i: (i, 0)),
            pl.BlockSpec((BN, 8), lambda i: (i, 0)),
            pl.BlockSpec((1, 1), lambda i: (0, 0)),
        ],
        out_shape=[
            jax.ShapeDtypeStruct((N_NODES, 8), jnp.float32),
            jax.ShapeDtypeStruct((N_NODES, 8), jnp.float32),
            jax.ShapeDtypeStruct((1, 1), jnp.float32),
        ],
        compiler_params=_TC_SEQ,
    )(h, w1_t, b1, w2_t8, b2_8, r1_t, rb1, r2_t8, rb2_8)


def _mpnn_block(xin, src, dst, ea, Wi, bi, Wh, bh, Wo, bo,
                srcj, rlist, jlist, starts, zeros_nd):
    node_dim = xin.shape[1]
    wix_t = Wi[:, :node_dim].T
    wie_t = Wi[:, node_dim:].T
    wh_t = Wh.T
    wox_t = Wo[:, :node_dim].T
    wom_t = Wo[:, node_dim:].T
    bi2 = bi.reshape(1, D)
    bh2 = bh.reshape(1, D)
    bo2 = bo.reshape(1, D)

    xw = _tc_node_matmul(xin, wix_t)
    s0 = _sc_gather_spmem(xw, src)
    h0b, gpad = _tc_edge_input(s0, ea, wie_t, bi2, bh2, wh_t)
    for k in range(DEPTH - 1):
        parts = _sc_scatter_add(gpad, dst, zeros_nd)
        pacc = _tc_partial_sum(parts)
        sr = _sc_combine(pacc, src, gpad, srcj, rlist, jlist, starts)
        if k < DEPTH - 2:
            gpad = _tc_edge_iter(h0b, sr, wh_t)
        else:
            h_fin = _tc_edge_last(h0b, sr)
    parts_m = _sc_scatter_add(h_fin, dst, zeros_nd)
    return _tc_readout(parts_m, xin, wox_t, wom_t, bo2)


def kernel(x, edge_index, edge_attr, pka_labels,
           g1_Wi, g1_bi, g1_Wh, g1_bh, g1_Wo, g1_bo,
           g3_Wi, g3_bi, g3_Wh, g3_bh, g3_Wo, g3_bo,
           cls_W1, cls_b1, cls_W2, cls_b2,
           reg_W1, reg_b1, reg_W2, reg_b2):
    src = edge_index[0]
    dst = edge_index[1]
    E = N_EDGES

    mult = jnp.max(edge_index).astype(jnp.int32) + 1
    keys = src.astype(jnp.int32) * mult + dst.astype(jnp.int32)
    rkeys = dst.astype(jnp.int32) * mult + src.astype(jnp.int32)
    order = jnp.argsort(keys, stable=True)
    sk = keys[order]
    pos = jnp.searchsorted(sk, rkeys)
    pos_c = jnp.clip(pos, 0, E - 1)
    match = sk[pos_c] == rkeys
    rev = jnp.where(match, order[pos_c], -1).astype(jnp.int32)
    revz = jnp.where(rev < 0, E, rev).astype(jnp.int32)

    valid = rev >= 0
    K = jnp.sum(valid.astype(jnp.int32))
    perm = jnp.argsort(jnp.logical_not(valid), stable=True)
    ar = jnp.arange(E, dtype=jnp.int32)
    inK = ar < K
    jcore = jnp.where(inK, perm.astype(jnp.int32), E)
    pad = jnp.full((CB,), E, jnp.int32)
    jlist = jnp.concatenate([jcore, pad])
    srcj = jnp.concatenate(
        [jnp.where(inK, src[perm], 0).astype(jnp.int32),
         jnp.zeros((CB,), jnp.int32)])
    rlist = jnp.concatenate(
        [jnp.where(inK, revz[perm], E).astype(jnp.int32), pad])
    starts = jnp.searchsorted(
        jcore, jnp.arange(33, dtype=jnp.int32) * EPW).astype(jnp.int32)
    starts = jnp.concatenate([starts, jnp.zeros((7,), jnp.int32)])

    zeros_nd = jnp.zeros((N_NODES, D), jnp.float32)

    h_static = _mpnn_block(x, src, dst, edge_attr,
                           g1_Wi, g1_bi, g1_Wh, g1_bh, g1_Wo, g1_bo,
                           srcj, rlist, jlist, starts, zeros_nd)
    h_cur = _mpnn_block(h_static, src, dst, edge_attr,
                        g3_Wi, g3_bi, g3_Wh, g3_bh, g3_Wo, g3_bo,
                        srcj, rlist, jlist, starts, zeros_nd)

    w2_t8 = jnp.zeros((D, 8), jnp.float32).at[:, :2].set(cls_W2.T)
    b2_8 = jnp.zeros((1, 8), jnp.float32).at[0, :2].set(cls_b2)
    r2_t8 = jnp.zeros((D, 8), jnp.float32).at[:, :1].set(reg_W2.T)
    rb2_8 = jnp.zeros((1, 8), jnp.float32).at[0, :1].set(reg_b2)

    logits8, pka8, lsum = _tc_heads(
        h_cur, cls_W1.T, cls_b1.reshape(1, D), w2_t8, b2_8,
        reg_W1.T, reg_b1.reshape(1, D), r2_t8, rb2_8)

    logits = logits8[:, :2]
    pka_raw = pka8[:, 0]
    loss_cla = lsum[0, 0] / N_NODES
    return (logits, pka_raw, 0.5 * loss_cla, loss_cla,
            jnp.array(0.0, jnp.float32))
